# Initial kernel scaffold; baseline (speedup 1.0000x reference)
import jax
import jax.numpy as jnp
from jax.experimental import pallas as pl


def kernel(x, emb, Wr, W1, b1, W2, b2, Wv, bv):
    raise NotImplementedError("write your pallas kernel here")


# speedup vs baseline: 1.1936x; 1.1936x over previous
import functools

import jax
import jax.numpy as jnp
from jax import lax
from jax.experimental import pallas as pl
from jax.experimental.pallas import tpu as pltpu
from jax.experimental.pallas import tpu_sc as plsc

F32 = jnp.float32


def _emb_gather(ids, emb):
    T = ids.shape[0]
    D = emb.shape[1]
    info = plsc.get_sparse_core_info()
    NW = info.num_cores * info.num_subcores
    b_per_w = T // NW
    mesh = plsc.VectorSubcoreMesh(core_axis_name="c", subcore_axis_name="s")

    @functools.partial(
        pl.kernel,
        mesh=mesh,
        out_type=jax.ShapeDtypeStruct((T, D), F32),
        scratch_types=[
            pltpu.VMEM((b_per_w,), jnp.int32),
            pltpu.VMEM((b_per_w, D), F32),
            pltpu.SemaphoreType.DMA,
        ],
    )
    def k(ids_hbm, table_hbm, out_hbm, idx_v, rows_v, sem):
        wid = lax.axis_index("s") * info.num_cores + lax.axis_index("c")
        base = wid * b_per_w
        pltpu.sync_copy(ids_hbm.at[pl.ds(base, b_per_w)], idx_v)
        pltpu.async_copy(table_hbm.at[idx_v], rows_v, sem).wait()
        pltpu.sync_copy(rows_v, out_hbm.at[pl.ds(base, b_per_w)])

    return k(ids, emb)


def _router_body(flat_ref, wr_ref, comb_ref, lb_ref):
    flat = flat_ref[...]
    logits = jnp.dot(flat, wr_ref[...], preferred_element_type=F32)
    lane = lax.broadcasted_iota(jnp.int32, logits.shape, 1)
    valid = lane < 8
    logits = jnp.where(valid, logits, -1e30)
    m = jnp.max(logits, axis=1, keepdims=True)
    ex = jnp.where(valid, jnp.exp(logits - m), 0.0)
    probs = ex / jnp.sum(ex, axis=1, keepdims=True)
    pm1 = jnp.where(valid, probs, -1.0)
    m1 = jnp.max(pm1, axis=1, keepdims=True)
    i1 = jnp.min(jnp.where(pm1 == m1, lane, 999), axis=1, keepdims=True)
    is1 = lane == i1
    pm2 = jnp.where(is1, -1.0, pm1)
    m2 = jnp.max(pm2, axis=1, keepdims=True)
    i2 = jnp.min(jnp.where(pm2 == m2, lane, 999), axis=1, keepdims=True)
    is2 = lane == i2
    denom = m1 + m2 + 1e-9
    comb = jnp.where(is1, m1 / denom, 0.0) + jnp.where(is2, m2 / denom, 0.0)
    comb_ref[...] = comb
    pmean = jnp.mean(probs, axis=0)
    frac = jnp.mean(is1.astype(F32) + is2.astype(F32), axis=0)
    lb_ref[0, 0] = 8.0 * jnp.sum(frac * pmean)


def _expert_body(flat_ref, w1_ref, b1_ref, w2_ref, b2_ref, comb_ref, out_ref):
    e = pl.program_id(1)
    x = flat_ref[...]
    h = jnp.dot(x, w1_ref[0], preferred_element_type=F32) + b1_ref[0]
    h = jax.nn.gelu(h)
    y = jnp.dot(h, w2_ref[0], preferred_element_type=F32) + b2_ref[0]
    lane = lax.broadcasted_iota(jnp.int32, comb_ref.shape, 1)
    coef = jnp.sum(jnp.where(lane == e, comb_ref[...], 0.0), axis=1,
                   keepdims=True)
    contrib = y * coef

    @pl.when(e == 0)
    def _():
        out_ref[...] = contrib

    @pl.when(e > 0)
    def _():
        out_ref[...] += contrib


def _vocab_body(out_ref, wv_ref, bv_ref, logits_ref):
    logits_ref[...] = (
        jnp.dot(out_ref[...], wv_ref[...], preferred_element_type=F32)
        + bv_ref[...]
    )


def kernel(x, emb, Wr, W1, b1, W2, b2, Wv, bv):
    Bv, S = x.shape
    V, D = emb.shape
    E, _, F = W1.shape
    T = Bv * S

    ids = x.reshape(T).astype(jnp.int32)
    flat = _emb_gather(ids, emb)

    Wr_pad = jnp.pad(Wr, ((0, 0), (0, 128 - E)))
    comb, lb = pl.pallas_call(
        _router_body,
        out_shape=[
            jax.ShapeDtypeStruct((T, 128), F32),
            jax.ShapeDtypeStruct((1, 1), F32),
        ],
        out_specs=[
            pl.BlockSpec((T, 128), lambda: (0, 0)),
            pl.BlockSpec(memory_space=pltpu.SMEM),
        ],
    )(flat, Wr_pad)

    TB = 512
    out = pl.pallas_call(
        _expert_body,
        grid=(T // TB, E),
        in_specs=[
            pl.BlockSpec((TB, D), lambda t, e: (t, 0)),
            pl.BlockSpec((1, D, F), lambda t, e: (e, 0, 0)),
            pl.BlockSpec((1, 1, F), lambda t, e: (e, 0, 0)),
            pl.BlockSpec((1, F, D), lambda t, e: (e, 0, 0)),
            pl.BlockSpec((1, 1, D), lambda t, e: (e, 0, 0)),
            pl.BlockSpec((TB, 128), lambda t, e: (t, 0)),
        ],
        out_specs=pl.BlockSpec((TB, D), lambda t, e: (t, 0)),
        out_shape=jax.ShapeDtypeStruct((T, D), F32),
    )(flat, W1, b1.reshape(E, 1, F), W2, b2.reshape(E, 1, D), comb)

    VB = 1280
    logits = pl.pallas_call(
        _vocab_body,
        grid=(V // VB,),
        in_specs=[
            pl.BlockSpec((T, D), lambda v: (0, 0)),
            pl.BlockSpec((D, VB), lambda v: (0, v)),
            pl.BlockSpec((1, VB), lambda v: (0, v)),
        ],
        out_specs=pl.BlockSpec((T, VB), lambda v: (0, v)),
        out_shape=jax.ShapeDtypeStruct((T, V), F32),
    )(out, Wv, bv.reshape(1, V))

    return logits.reshape(Bv, S, V), lb.reshape(())


# speedup vs baseline: 1.3536x; 1.1340x over previous
import functools

import jax
import jax.numpy as jnp
from jax import lax
from jax.experimental import pallas as pl
from jax.experimental.pallas import tpu as pltpu
from jax.experimental.pallas import tpu_sc as plsc

F32 = jnp.float32
BLK = 256
NBLK = 24


def _emb_gather(ids, emb):
    T = ids.shape[0]
    D = emb.shape[1]
    info = plsc.get_sparse_core_info()
    NW = info.num_cores * info.num_subcores
    b_per_w = T // NW
    mesh = plsc.VectorSubcoreMesh(core_axis_name="c", subcore_axis_name="s")

    @functools.partial(
        pl.kernel,
        mesh=mesh,
        out_type=jax.ShapeDtypeStruct((T, D), F32),
        scratch_types=[
            pltpu.VMEM((b_per_w,), jnp.int32),
            pltpu.VMEM((b_per_w, D), F32),
            pltpu.SemaphoreType.DMA,
        ],
    )
    def k(ids_hbm, table_hbm, out_hbm, idx_v, rows_v, sem):
        wid = lax.axis_index("s") * info.num_cores + lax.axis_index("c")
        base = wid * b_per_w
        pltpu.sync_copy(ids_hbm.at[pl.ds(base, b_per_w)], idx_v)
        pltpu.async_copy(table_hbm.at[idx_v], rows_v, sem).wait()
        pltpu.sync_copy(rows_v, out_hbm.at[pl.ds(base, b_per_w)])

    return k(ids, emb)


def _dispatch_scatter(flat, inv0, inv1, cap):
    T, D = flat.shape
    info = plsc.get_sparse_core_info()
    NW = info.num_cores * info.num_subcores
    b_per_w = T // NW
    mesh = plsc.VectorSubcoreMesh(core_axis_name="c", subcore_axis_name="s")

    @functools.partial(
        pl.kernel,
        mesh=mesh,
        out_type=jax.ShapeDtypeStruct((cap, D), F32),
        scratch_types=[
            pltpu.VMEM((b_per_w,), jnp.int32),
            pltpu.VMEM((b_per_w, D), F32),
            pltpu.SemaphoreType.DMA,
        ],
    )
    def k(flat_hbm, inv0_hbm, inv1_hbm, xs_hbm, idx_v, rows_v, sem):
        wid = lax.axis_index("s") * info.num_cores + lax.axis_index("c")
        base = wid * b_per_w
        pltpu.sync_copy(flat_hbm.at[pl.ds(base, b_per_w)], rows_v)
        pltpu.sync_copy(inv0_hbm.at[pl.ds(base, b_per_w)], idx_v)
        pltpu.async_copy(rows_v, xs_hbm.at[idx_v], sem).wait()
        pltpu.sync_copy(inv1_hbm.at[pl.ds(base, b_per_w)], idx_v)
        pltpu.async_copy(rows_v, xs_hbm.at[idx_v], sem).wait()

    return k(flat, inv0, inv1)


def _combine_gather(yw, inv0, inv1):
    T = inv0.shape[0]
    D = yw.shape[1]
    info = plsc.get_sparse_core_info()
    NW = info.num_cores * info.num_subcores
    b_per_w = T // NW
    mesh = plsc.VectorSubcoreMesh(core_axis_name="c", subcore_axis_name="s")

    @functools.partial(
        pl.kernel,
        mesh=mesh,
        out_type=[
            jax.ShapeDtypeStruct((T, D), F32),
            jax.ShapeDtypeStruct((T, D), F32),
        ],
        scratch_types=[
            pltpu.VMEM((b_per_w,), jnp.int32),
            pltpu.VMEM((b_per_w, D), F32),
            pltpu.SemaphoreType.DMA,
        ],
    )
    def k(yw_hbm, inv0_hbm, inv1_hbm, ga_hbm, gb_hbm, idx_v, rows_v, sem):
        wid = lax.axis_index("s") * info.num_cores + lax.axis_index("c")
        base = wid * b_per_w
        pltpu.sync_copy(inv0_hbm.at[pl.ds(base, b_per_w)], idx_v)
        pltpu.async_copy(yw_hbm.at[idx_v], rows_v, sem).wait()
        pltpu.sync_copy(rows_v, ga_hbm.at[pl.ds(base, b_per_w)])
        pltpu.sync_copy(inv1_hbm.at[pl.ds(base, b_per_w)], idx_v)
        pltpu.async_copy(yw_hbm.at[idx_v], rows_v, sem).wait()
        pltpu.sync_copy(rows_v, gb_hbm.at[pl.ds(base, b_per_w)])

    return k(yw, inv0, inv1)


def _router_meta_body(flat_ref, wr_ref, invs_ref, gcol_ref, meta_ref, lb_ref):
    flat = flat_ref[...]
    T = flat.shape[0]
    logits = jnp.dot(flat, wr_ref[...], preferred_element_type=F32)
    lane = lax.broadcasted_iota(jnp.int32, logits.shape, 1)
    valid = lane < 8
    logits = jnp.where(valid, logits, -1e30)
    m = jnp.max(logits, axis=1, keepdims=True)
    ex = jnp.where(valid, jnp.exp(logits - m), 0.0)
    probs = ex / jnp.sum(ex, axis=1, keepdims=True)
    pm1 = jnp.where(valid, probs, -1.0)
    m1 = jnp.max(pm1, axis=1, keepdims=True)
    i1 = jnp.min(jnp.where(pm1 == m1, lane, 999), axis=1, keepdims=True)
    is1 = lane == i1
    pm2 = jnp.where(is1, -1.0, pm1)
    m2 = jnp.max(pm2, axis=1, keepdims=True)
    i2 = jnp.min(jnp.where(pm2 == m2, lane, 999), axis=1, keepdims=True)
    is2 = lane == i2
    denom = m1 + m2 + 1e-9
    g1 = m1 / denom
    g2 = m2 / denom
    mask = is1 | is2
    maskf = mask.astype(F32)

    r0 = lax.broadcasted_iota(jnp.int32, (128, 128), 0)
    c0 = lax.broadcasted_iota(jnp.int32, (128, 128), 1)
    lt = (r0 >= c0).astype(F32)
    uts = (r0 < c0).astype(F32)
    chunks = []
    running = jnp.zeros((1, 128), F32)
    for c in range(T // 128):
        blk = maskf[c * 128:(c + 1) * 128, :]
        cumc = jnp.dot(lt, blk, preferred_element_type=F32) + running
        running = cumc[127:128, :]
        chunks.append(cumc)
    cum = jnp.concatenate(chunks, axis=0)
    counts = running
    ci = counts.astype(jnp.int32)
    nb = (ci + (BLK - 1)) // BLK
    nbf = nb.astype(F32)
    baseblk = jnp.dot(nbf, uts, preferred_element_type=F32)
    basef = BLK * baseblk
    pos = basef + cum - 1.0

    colrank = jnp.dot(maskf, uts, preferred_element_type=F32)
    sel0 = maskf * (colrank == 0.0).astype(F32)
    sel1 = maskf * (colrank == 1.0).astype(F32)
    inv0 = jnp.sum(sel0 * pos, axis=1, keepdims=True).astype(jnp.int32)
    inv1 = jnp.sum(sel1 * pos, axis=1, keepdims=True).astype(jnp.int32)
    glow = jnp.where(i1 < i2, g1, g2)
    ghigh = jnp.where(i1 < i2, g2, g1)
    invs_ref[...] = jnp.where(lane == 0, inv0,
                              jnp.where(lane == 1, inv1, 0))
    gcol_ref[...] = jnp.where(lane == 0, glow,
                              jnp.where(lane == 1, ghigh, 0.0))

    jrow = lax.broadcasted_iota(jnp.int32, (32, 128), 0).astype(F32)
    lane32 = lax.broadcasted_iota(jnp.int32, (32, 128), 1)
    bb = jnp.broadcast_to(baseblk, (32, 128))
    nbb = jnp.broadcast_to(nbf, (32, 128))
    active_e = ((jrow >= bb) & (jrow < bb + nbb)).astype(F32)
    bexpf = jnp.sum(active_e * lane32.astype(F32), axis=1, keepdims=True)
    nact = jnp.sum(active_e, axis=1, keepdims=True)
    totb = jnp.sum(nbf)
    le = jnp.max(jnp.where(ci > 0, lane[0:1, :], -1))
    bexp = jnp.where(nact > 0, bexpf, le.astype(F32)).astype(jnp.int32)
    act = (jrow[:, 0:1] < totb).astype(jnp.int32)
    meta_ref[...] = jnp.where(lane32 == 0, bexp,
                              jnp.where(lane32 == 1, act, 0))

    pmean = jnp.mean(probs, axis=0)
    frac = jnp.mean(maskf, axis=0)
    lb_ref[0, 0] = 8.0 * jnp.sum(frac * pmean)


def _ffn_body(bexp_ref, act_ref, xs_ref, w1_ref, b1_ref, w2_ref, b2_ref,
              yw_ref):
    j = pl.program_id(0)

    @pl.when(act_ref[j] > 0)
    def _():
        x = xs_ref[...]
        h = jnp.dot(x, w1_ref[0], preferred_element_type=F32) + b1_ref[0]
        h = jax.nn.gelu(h)
        yw_ref[...] = (
            jnp.dot(h, w2_ref[0], preferred_element_type=F32) + b2_ref[0]
        )


def _combine_body(ga_ref, gb_ref, gcol_ref, out_ref):
    out_ref[...] = (ga_ref[...] * gcol_ref[:, 0:1]
                    + gb_ref[...] * gcol_ref[:, 1:2])


def _vocab_body(out_ref, wv_ref, bv_ref, logits_ref):
    logits_ref[...] = (
        jnp.dot(out_ref[...], wv_ref[...], preferred_element_type=F32)
        + bv_ref[...]
    )


def kernel(x, emb, Wr, W1, b1, W2, b2, Wv, bv):
    Bv, S = x.shape
    V, D = emb.shape
    E, _, F = W1.shape
    T = Bv * S
    CAP = NBLK * BLK

    ids = x.reshape(T).astype(jnp.int32)
    flat = _emb_gather(ids, emb)

    Wr_pad = jnp.pad(Wr, ((0, 0), (0, 128 - E)))
    invs, gcol, meta, lb = pl.pallas_call(
        _router_meta_body,
        out_shape=[
            jax.ShapeDtypeStruct((T, 128), jnp.int32),
            jax.ShapeDtypeStruct((T, 128), F32),
            jax.ShapeDtypeStruct((32, 128), jnp.int32),
            jax.ShapeDtypeStruct((1, 1), F32),
        ],
        out_specs=[
            pl.BlockSpec((T, 128), lambda: (0, 0)),
            pl.BlockSpec((T, 128), lambda: (0, 0)),
            pl.BlockSpec((32, 128), lambda: (0, 0)),
            pl.BlockSpec(memory_space=pltpu.SMEM),
        ],
    )(flat, Wr_pad)

    inv0 = invs[:, 0]
    inv1 = invs[:, 1]
    bexp = meta[:NBLK, 0]
    act = meta[:NBLK, 1]

    xs = _dispatch_scatter(flat, inv0, inv1, CAP)

    yw = pl.pallas_call(
        _ffn_body,
        grid_spec=pltpu.PrefetchScalarGridSpec(
            num_scalar_prefetch=2,
            grid=(NBLK,),
            in_specs=[
                pl.BlockSpec((BLK, D), lambda j, be, ac: (j, 0)),
                pl.BlockSpec((1, D, F), lambda j, be, ac: (be[j], 0, 0)),
                pl.BlockSpec((1, 1, F), lambda j, be, ac: (be[j], 0, 0)),
                pl.BlockSpec((1, F, D), lambda j, be, ac: (be[j], 0, 0)),
                pl.BlockSpec((1, 1, D), lambda j, be, ac: (be[j], 0, 0)),
            ],
            out_specs=pl.BlockSpec((BLK, D), lambda j, be, ac: (j, 0)),
        ),
        out_shape=jax.ShapeDtypeStruct((CAP, D), F32),
    )(bexp, act, xs, W1, b1.reshape(E, 1, F), W2, b2.reshape(E, 1, D))

    ga, gb = _combine_gather(yw, inv0, inv1)

    out = pl.pallas_call(
        _combine_body,
        out_shape=jax.ShapeDtypeStruct((T, D), F32),
    )(ga, gb, gcol)

    VB = 1280
    logits = pl.pallas_call(
        _vocab_body,
        grid=(V // VB,),
        in_specs=[
            pl.BlockSpec((T, D), lambda v: (0, 0)),
            pl.BlockSpec((D, VB), lambda v: (0, v)),
            pl.BlockSpec((1, VB), lambda v: (0, v)),
        ],
        out_specs=pl.BlockSpec((T, VB), lambda v: (0, v)),
        out_shape=jax.ShapeDtypeStruct((T, V), F32),
    )(out, Wv, bv.reshape(1, V))

    return logits.reshape(Bv, S, V), lb.reshape(())


# speedup vs baseline: 1.4016x; 1.0355x over previous
import functools

import jax
import jax.numpy as jnp
from jax import lax
from jax.experimental import pallas as pl
from jax.experimental.pallas import tpu as pltpu
from jax.experimental.pallas import tpu_sc as plsc

F32 = jnp.float32
BF16 = jnp.bfloat16
BLK = 512
NBLK = 16


def _emb_gather(ids, emb):
    T = ids.shape[0]
    D = emb.shape[1]
    info = plsc.get_sparse_core_info()
    NW = info.num_cores * info.num_subcores
    b_per_w = T // NW
    mesh = plsc.VectorSubcoreMesh(core_axis_name="c", subcore_axis_name="s")

    @functools.partial(
        pl.kernel,
        mesh=mesh,
        out_type=jax.ShapeDtypeStruct((T, D), F32),
        scratch_types=[
            pltpu.VMEM((b_per_w,), jnp.int32),
            pltpu.VMEM((b_per_w, D), F32),
            pltpu.SemaphoreType.DMA,
        ],
    )
    def k(ids_hbm, table_hbm, out_hbm, idx_v, rows_v, sem):
        wid = lax.axis_index("s") * info.num_cores + lax.axis_index("c")
        base = wid * b_per_w
        pltpu.sync_copy(ids_hbm.at[pl.ds(base, b_per_w)], idx_v)
        pltpu.async_copy(table_hbm.at[idx_v], rows_v, sem).wait()
        pltpu.sync_copy(rows_v, out_hbm.at[pl.ds(base, b_per_w)])

    return k(ids, emb)


def _dispatch_scatter(flat, inv0, inv1, cap):
    T, D = flat.shape
    info = plsc.get_sparse_core_info()
    NW = info.num_cores * info.num_subcores
    b_per_w = T // NW
    mesh = plsc.VectorSubcoreMesh(core_axis_name="c", subcore_axis_name="s")

    @functools.partial(
        pl.kernel,
        mesh=mesh,
        out_type=jax.ShapeDtypeStruct((cap, D), F32),
        scratch_types=[
            pltpu.VMEM((b_per_w,), jnp.int32),
            pltpu.VMEM((b_per_w, D), F32),
            pltpu.SemaphoreType.DMA,
        ],
    )
    def k(flat_hbm, inv0_hbm, inv1_hbm, xs_hbm, idx_v, rows_v, sem):
        wid = lax.axis_index("s") * info.num_cores + lax.axis_index("c")
        base = wid * b_per_w
        pltpu.sync_copy(flat_hbm.at[pl.ds(base, b_per_w)], rows_v)
        pltpu.sync_copy(inv0_hbm.at[pl.ds(base, b_per_w)], idx_v)
        pltpu.async_copy(rows_v, xs_hbm.at[idx_v], sem).wait()
        pltpu.sync_copy(inv1_hbm.at[pl.ds(base, b_per_w)], idx_v)
        pltpu.async_copy(rows_v, xs_hbm.at[idx_v], sem).wait()

    return k(flat, inv0, inv1)


def _combine_gather(yw, inv0, inv1):
    T = inv0.shape[0]
    D = yw.shape[1]
    info = plsc.get_sparse_core_info()
    NW = info.num_cores * info.num_subcores
    b_per_w = T // NW
    mesh = plsc.VectorSubcoreMesh(core_axis_name="c", subcore_axis_name="s")

    @functools.partial(
        pl.kernel,
        mesh=mesh,
        out_type=[
            jax.ShapeDtypeStruct((T, D), F32),
            jax.ShapeDtypeStruct((T, D), F32),
        ],
        scratch_types=[
            pltpu.VMEM((b_per_w,), jnp.int32),
            pltpu.VMEM((b_per_w, D), F32),
            pltpu.SemaphoreType.DMA,
        ],
    )
    def k(yw_hbm, inv0_hbm, inv1_hbm, ga_hbm, gb_hbm, idx_v, rows_v, sem):
        wid = lax.axis_index("s") * info.num_cores + lax.axis_index("c")
        base = wid * b_per_w
        pltpu.sync_copy(inv0_hbm.at[pl.ds(base, b_per_w)], idx_v)
        pltpu.async_copy(yw_hbm.at[idx_v], rows_v, sem).wait()
        pltpu.sync_copy(rows_v, ga_hbm.at[pl.ds(base, b_per_w)])
        pltpu.sync_copy(inv1_hbm.at[pl.ds(base, b_per_w)], idx_v)
        pltpu.async_copy(yw_hbm.at[idx_v], rows_v, sem).wait()
        pltpu.sync_copy(rows_v, gb_hbm.at[pl.ds(base, b_per_w)])

    return k(yw, inv0, inv1)


def _router_meta_body(flat_ref, wr_ref, invs_ref, gcol_ref, meta_ref, lb_ref):
    flat = flat_ref[...]
    T = flat.shape[0]
    logits = jnp.dot(flat, wr_ref[...], preferred_element_type=F32)
    lane = lax.broadcasted_iota(jnp.int32, logits.shape, 1)
    valid = lane < 8
    logits = jnp.where(valid, logits, -1e30)
    m = jnp.max(logits, axis=1, keepdims=True)
    ex = jnp.where(valid, jnp.exp(logits - m), 0.0)
    probs = ex / jnp.sum(ex, axis=1, keepdims=True)
    pm1 = jnp.where(valid, probs, -1.0)
    m1 = jnp.max(pm1, axis=1, keepdims=True)
    i1 = jnp.min(jnp.where(pm1 == m1, lane, 999), axis=1, keepdims=True)
    is1 = lane == i1
    pm2 = jnp.where(is1, -1.0, pm1)
    m2 = jnp.max(pm2, axis=1, keepdims=True)
    i2 = jnp.min(jnp.where(pm2 == m2, lane, 999), axis=1, keepdims=True)
    is2 = lane == i2
    denom = m1 + m2 + 1e-9
    g1 = m1 / denom
    g2 = m2 / denom
    mask = is1 | is2
    maskf = mask.astype(F32)

    r0 = lax.broadcasted_iota(jnp.int32, (128, 128), 0)
    c0 = lax.broadcasted_iota(jnp.int32, (128, 128), 1)
    lt = (r0 >= c0).astype(F32)
    uts = (r0 < c0).astype(F32)
    chunks = []
    running = jnp.zeros((1, 128), F32)
    for c in range(T // 128):
        blk = maskf[c * 128:(c + 1) * 128, :]
        cumc = jnp.dot(lt, blk, preferred_element_type=F32) + running
        running = cumc[127:128, :]
        chunks.append(cumc)
    cum = jnp.concatenate(chunks, axis=0)
    counts = running
    ci = counts.astype(jnp.int32)
    nb = (ci + (BLK - 1)) // BLK
    nbf = nb.astype(F32)
    baseblk = jnp.dot(nbf, uts, preferred_element_type=F32)
    basef = BLK * baseblk
    pos = basef + cum - 1.0

    colrank = jnp.dot(maskf, uts, preferred_element_type=F32)
    sel0 = maskf * (colrank == 0.0).astype(F32)
    sel1 = maskf * (colrank == 1.0).astype(F32)
    inv0 = jnp.sum(sel0 * pos, axis=1, keepdims=True).astype(jnp.int32)
    inv1 = jnp.sum(sel1 * pos, axis=1, keepdims=True).astype(jnp.int32)
    glow = jnp.where(i1 < i2, g1, g2)
    ghigh = jnp.where(i1 < i2, g2, g1)
    invs_ref[...] = jnp.where(lane == 0, inv0,
                              jnp.where(lane == 1, inv1, 0))
    gcol_ref[...] = jnp.where(lane == 0, glow,
                              jnp.where(lane == 1, ghigh, 0.0))

    jrow = lax.broadcasted_iota(jnp.int32, (32, 128), 0).astype(F32)
    lane32 = lax.broadcasted_iota(jnp.int32, (32, 128), 1)
    bb = jnp.broadcast_to(baseblk, (32, 128))
    nbb = jnp.broadcast_to(nbf, (32, 128))
    active_e = ((jrow >= bb) & (jrow < bb + nbb)).astype(F32)
    bexpf = jnp.sum(active_e * lane32.astype(F32), axis=1, keepdims=True)
    nact = jnp.sum(active_e, axis=1, keepdims=True)
    totb = jnp.sum(nbf)
    le = jnp.max(jnp.where(ci > 0, lane[0:1, :], -1))
    bexp = jnp.where(nact > 0, bexpf, le.astype(F32)).astype(jnp.int32)
    act = (jrow[:, 0:1] < totb).astype(jnp.int32)
    meta_ref[...] = jnp.where(lane32 == 0, bexp,
                              jnp.where(lane32 == 1, act, 0))

    pmean = jnp.mean(probs, axis=0)
    frac = jnp.mean(maskf, axis=0)
    lb_ref[0, 0] = 8.0 * jnp.sum(frac * pmean)


def _ffn_body(bexp_ref, act_ref, xs_ref, w1_ref, b1_ref, w2_ref, b2_ref,
              yw_ref):
    j = pl.program_id(0)

    @pl.when(act_ref[j] > 0)
    def _():
        x = xs_ref[...]
        h = jnp.dot(x, w1_ref[0], preferred_element_type=F32) + b1_ref[0]
        h = jax.nn.gelu(h)
        yw_ref[...] = (
            jnp.dot(h, w2_ref[0], preferred_element_type=F32) + b2_ref[0]
        )


def _combine_body(ga_ref, gb_ref, gcol_ref, out_ref):
    out_ref[...] = (ga_ref[...] * gcol_ref[:, 0:1]
                    + gb_ref[...] * gcol_ref[:, 1:2])


def _vocab_body(out_ref, wv_ref, bv_ref, logits_ref):
    logits_ref[...] = (
        jnp.dot(out_ref[...], wv_ref[...], preferred_element_type=F32)
        + bv_ref[...]
    )


def kernel(x, emb, Wr, W1, b1, W2, b2, Wv, bv):
    Bv, S = x.shape
    V, D = emb.shape
    E, _, F = W1.shape
    T = Bv * S
    CAP = NBLK * BLK

    ids = x.reshape(T).astype(jnp.int32)
    flat = _emb_gather(ids, emb)

    Wr_pad = jnp.pad(Wr, ((0, 0), (0, 128 - E)))
    invs, gcol, meta, lb = pl.pallas_call(
        _router_meta_body,
        out_shape=[
            jax.ShapeDtypeStruct((T, 128), jnp.int32),
            jax.ShapeDtypeStruct((T, 128), F32),
            jax.ShapeDtypeStruct((32, 128), jnp.int32),
            jax.ShapeDtypeStruct((1, 1), F32),
        ],
        out_specs=[
            pl.BlockSpec((T, 128), lambda: (0, 0)),
            pl.BlockSpec((T, 128), lambda: (0, 0)),
            pl.BlockSpec((32, 128), lambda: (0, 0)),
            pl.BlockSpec(memory_space=pltpu.SMEM),
        ],
    )(flat, Wr_pad)

    inv0 = invs[:, 0]
    inv1 = invs[:, 1]
    bexp = meta[:NBLK, 0]
    act = meta[:NBLK, 1]

    xs = _dispatch_scatter(flat, inv0, inv1, CAP)

    yw = pl.pallas_call(
        _ffn_body,
        grid_spec=pltpu.PrefetchScalarGridSpec(
            num_scalar_prefetch=2,
            grid=(NBLK,),
            in_specs=[
                pl.BlockSpec((BLK, D), lambda j, be, ac: (j, 0)),
                pl.BlockSpec((1, D, F), lambda j, be, ac: (be[j], 0, 0)),
                pl.BlockSpec((1, 1, F), lambda j, be, ac: (be[j], 0, 0)),
                pl.BlockSpec((1, F, D), lambda j, be, ac: (be[j], 0, 0)),
                pl.BlockSpec((1, 1, D), lambda j, be, ac: (be[j], 0, 0)),
            ],
            out_specs=pl.BlockSpec((BLK, D), lambda j, be, ac: (j, 0)),
        ),
        out_shape=jax.ShapeDtypeStruct((CAP, D), F32),
    )(bexp, act, xs, W1, b1.reshape(E, 1, F), W2, b2.reshape(E, 1, D))

    ga, gb = _combine_gather(yw, inv0, inv1)

    out = pl.pallas_call(
        _combine_body,
        out_shape=jax.ShapeDtypeStruct((T, D), F32),
    )(ga, gb, gcol)

    VB = 1280
    logits = pl.pallas_call(
        _vocab_body,
        grid=(V // VB,),
        in_specs=[
            pl.BlockSpec((T, D), lambda v: (0, 0)),
            pl.BlockSpec((D, VB), lambda v: (0, v)),
            pl.BlockSpec((1, VB), lambda v: (0, v)),
        ],
        out_specs=pl.BlockSpec((T, VB), lambda v: (0, v)),
        out_shape=jax.ShapeDtypeStruct((T, V), F32),
    )(out, Wv, bv.reshape(1, V))

    return logits.reshape(Bv, S, V), lb.reshape(())


# speedup vs baseline: 1.4254x; 1.0170x over previous
import functools

import jax
import jax.numpy as jnp
from jax import lax
from jax.experimental import pallas as pl
from jax.experimental.pallas import tpu as pltpu
from jax.experimental.pallas import tpu_sc as plsc

F32 = jnp.float32
BF16 = jnp.bfloat16
BLK = 512
NBLK = 16


def _emb_gather(ids, emb):
    T = ids.shape[0]
    D = emb.shape[1]
    info = plsc.get_sparse_core_info()
    NW = info.num_cores * info.num_subcores
    b_per_w = T // NW
    mesh = plsc.VectorSubcoreMesh(core_axis_name="c", subcore_axis_name="s")

    @functools.partial(
        pl.kernel,
        mesh=mesh,
        out_type=jax.ShapeDtypeStruct((T, D), F32),
        scratch_types=[
            pltpu.VMEM((b_per_w,), jnp.int32),
            pltpu.VMEM((b_per_w, D), F32),
            pltpu.SemaphoreType.DMA,
        ],
    )
    def k(ids_hbm, table_hbm, out_hbm, idx_v, rows_v, sem):
        wid = lax.axis_index("s") * info.num_cores + lax.axis_index("c")
        base = wid * b_per_w
        pltpu.sync_copy(ids_hbm.at[pl.ds(base, b_per_w)], idx_v)
        pltpu.async_copy(table_hbm.at[idx_v], rows_v, sem).wait()
        pltpu.sync_copy(rows_v, out_hbm.at[pl.ds(base, b_per_w)])

    return k(ids, emb)


def _dispatch_scatter(flat, inv0, inv1, cap):
    T, D = flat.shape
    info = plsc.get_sparse_core_info()
    NW = info.num_cores * info.num_subcores
    b_per_w = T // NW
    mesh = plsc.VectorSubcoreMesh(core_axis_name="c", subcore_axis_name="s")

    @functools.partial(
        pl.kernel,
        mesh=mesh,
        out_type=jax.ShapeDtypeStruct((cap, D), F32),
        scratch_types=[
            pltpu.VMEM((b_per_w,), jnp.int32),
            pltpu.VMEM((b_per_w, D), F32),
            pltpu.SemaphoreType.DMA,
        ],
    )
    def k(flat_hbm, inv0_hbm, inv1_hbm, xs_hbm, idx_v, rows_v, sem):
        wid = lax.axis_index("s") * info.num_cores + lax.axis_index("c")
        base = wid * b_per_w
        pltpu.sync_copy(flat_hbm.at[pl.ds(base, b_per_w)], rows_v)
        pltpu.sync_copy(inv0_hbm.at[pl.ds(base, b_per_w)], idx_v)
        pltpu.async_copy(rows_v, xs_hbm.at[idx_v], sem).wait()
        pltpu.sync_copy(inv1_hbm.at[pl.ds(base, b_per_w)], idx_v)
        pltpu.async_copy(rows_v, xs_hbm.at[idx_v], sem).wait()

    return k(flat, inv0, inv1)


def _combine_gather(yw, inv0, inv1):
    T = inv0.shape[0]
    D = yw.shape[1]
    info = plsc.get_sparse_core_info()
    NW = info.num_cores * info.num_subcores
    b_per_w = T // NW
    mesh = plsc.VectorSubcoreMesh(core_axis_name="c", subcore_axis_name="s")

    @functools.partial(
        pl.kernel,
        mesh=mesh,
        out_type=[
            jax.ShapeDtypeStruct((T, D), F32),
            jax.ShapeDtypeStruct((T, D), F32),
        ],
        scratch_types=[
            pltpu.VMEM((b_per_w,), jnp.int32),
            pltpu.VMEM((b_per_w, D), F32),
            pltpu.SemaphoreType.DMA,
        ],
    )
    def k(yw_hbm, inv0_hbm, inv1_hbm, ga_hbm, gb_hbm, idx_v, rows_v, sem):
        wid = lax.axis_index("s") * info.num_cores + lax.axis_index("c")
        base = wid * b_per_w
        pltpu.sync_copy(inv0_hbm.at[pl.ds(base, b_per_w)], idx_v)
        pltpu.async_copy(yw_hbm.at[idx_v], rows_v, sem).wait()
        pltpu.sync_copy(rows_v, ga_hbm.at[pl.ds(base, b_per_w)])
        pltpu.sync_copy(inv1_hbm.at[pl.ds(base, b_per_w)], idx_v)
        pltpu.async_copy(yw_hbm.at[idx_v], rows_v, sem).wait()
        pltpu.sync_copy(rows_v, gb_hbm.at[pl.ds(base, b_per_w)])

    return k(yw, inv0, inv1)


def _router_meta_body(flat_ref, wr_ref, invs_ref, gcol_ref, meta_ref, lb_ref):
    flat = flat_ref[...]
    T = flat.shape[0]
    logits = jnp.dot(flat, wr_ref[...], preferred_element_type=F32)
    lane = lax.broadcasted_iota(jnp.int32, logits.shape, 1)
    valid = lane < 8
    logits = jnp.where(valid, logits, -1e30)
    m = jnp.max(logits, axis=1, keepdims=True)
    ex = jnp.where(valid, jnp.exp(logits - m), 0.0)
    probs = ex / jnp.sum(ex, axis=1, keepdims=True)
    pm1 = jnp.where(valid, probs, -1.0)
    m1 = jnp.max(pm1, axis=1, keepdims=True)
    i1 = jnp.min(jnp.where(pm1 == m1, lane, 999), axis=1, keepdims=True)
    is1 = lane == i1
    pm2 = jnp.where(is1, -1.0, pm1)
    m2 = jnp.max(pm2, axis=1, keepdims=True)
    i2 = jnp.min(jnp.where(pm2 == m2, lane, 999), axis=1, keepdims=True)
    is2 = lane == i2
    denom = m1 + m2 + 1e-9
    g1 = m1 / denom
    g2 = m2 / denom
    mask = is1 | is2
    maskf = mask.astype(F32)

    r0 = lax.broadcasted_iota(jnp.int32, (128, 128), 0)
    c0 = lax.broadcasted_iota(jnp.int32, (128, 128), 1)
    lt = (r0 >= c0).astype(F32)
    uts = (r0 < c0).astype(F32)
    chunks = []
    running = jnp.zeros((1, 128), F32)
    for c in range(T // 128):
        blk = maskf[c * 128:(c + 1) * 128, :]
        cumc = jnp.dot(lt, blk, preferred_element_type=F32) + running
        running = cumc[127:128, :]
        chunks.append(cumc)
    cum = jnp.concatenate(chunks, axis=0)
    counts = running
    ci = counts.astype(jnp.int32)
    nb = (ci + (BLK - 1)) // BLK
    nbf = nb.astype(F32)
    baseblk = jnp.dot(nbf, uts, preferred_element_type=F32)
    basef = BLK * baseblk
    pos = basef + cum - 1.0

    colrank = jnp.dot(maskf, uts, preferred_element_type=F32)
    sel0 = maskf * (colrank == 0.0).astype(F32)
    sel1 = maskf * (colrank == 1.0).astype(F32)
    inv0 = jnp.sum(sel0 * pos, axis=1, keepdims=True).astype(jnp.int32)
    inv1 = jnp.sum(sel1 * pos, axis=1, keepdims=True).astype(jnp.int32)
    glow = jnp.where(i1 < i2, g1, g2)
    ghigh = jnp.where(i1 < i2, g2, g1)
    invs_ref[...] = jnp.where(lane == 0, inv0,
                              jnp.where(lane == 1, inv1, 0))
    gcol_ref[...] = jnp.where(lane == 0, glow,
                              jnp.where(lane == 1, ghigh, 0.0))

    jrow = lax.broadcasted_iota(jnp.int32, (32, 128), 0).astype(F32)
    lane32 = lax.broadcasted_iota(jnp.int32, (32, 128), 1)
    bb = jnp.broadcast_to(baseblk, (32, 128))
    nbb = jnp.broadcast_to(nbf, (32, 128))
    active_e = ((jrow >= bb) & (jrow < bb + nbb)).astype(F32)
    bexpf = jnp.sum(active_e * lane32.astype(F32), axis=1, keepdims=True)
    nact = jnp.sum(active_e, axis=1, keepdims=True)
    totb = jnp.sum(nbf)
    le = jnp.max(jnp.where(ci > 0, lane[0:1, :], -1))
    bexp = jnp.where(nact > 0, bexpf, le.astype(F32)).astype(jnp.int32)
    lst = jnp.minimum(jrow[:, 0:1], totb - 1.0).astype(jnp.int32)
    meta_ref[...] = jnp.where(lane32 == 0, bexp,
                              jnp.where(lane32 == 1, lst, 0))

    pmean = jnp.mean(probs, axis=0)
    frac = jnp.mean(maskf, axis=0)
    lb_ref[0, 0] = 8.0 * jnp.sum(frac * pmean)


def _ffn_body(bexp_ref, lst_ref, xs_ref, w1_ref, b1_ref, w2_ref, b2_ref,
              yw_ref):
    j = pl.program_id(0)

    @pl.when(lst_ref[j] == j)
    def _():
        x = xs_ref[...]
        h = jnp.dot(x, w1_ref[0], preferred_element_type=F32) + b1_ref[0]
        h = jax.nn.gelu(h)
        yw_ref[...] = (
            jnp.dot(h, w2_ref[0], preferred_element_type=F32) + b2_ref[0]
        )


def _combine_body(ga_ref, gb_ref, gcol_ref, out_ref):
    out_ref[...] = (ga_ref[...] * gcol_ref[:, 0:1]
                    + gb_ref[...] * gcol_ref[:, 1:2])


def _vocab_body(out_ref, wv_ref, bv_ref, logits_ref):
    logits_ref[...] = (
        jnp.dot(out_ref[...], wv_ref[...], preferred_element_type=F32)
        + bv_ref[...]
    )


def kernel(x, emb, Wr, W1, b1, W2, b2, Wv, bv):
    Bv, S = x.shape
    V, D = emb.shape
    E, _, F = W1.shape
    T = Bv * S
    CAP = NBLK * BLK

    ids = x.reshape(T).astype(jnp.int32)
    flat = _emb_gather(ids, emb)

    Wr_pad = jnp.pad(Wr, ((0, 0), (0, 128 - E)))
    invs, gcol, meta, lb = pl.pallas_call(
        _router_meta_body,
        out_shape=[
            jax.ShapeDtypeStruct((T, 128), jnp.int32),
            jax.ShapeDtypeStruct((T, 128), F32),
            jax.ShapeDtypeStruct((32, 128), jnp.int32),
            jax.ShapeDtypeStruct((1, 1), F32),
        ],
        out_specs=[
            pl.BlockSpec((T, 128), lambda: (0, 0)),
            pl.BlockSpec((T, 128), lambda: (0, 0)),
            pl.BlockSpec((32, 128), lambda: (0, 0)),
            pl.BlockSpec(memory_space=pltpu.SMEM),
        ],
    )(flat, Wr_pad)

    inv0 = invs[:, 0]
    inv1 = invs[:, 1]
    bexp = meta[:NBLK, 0]
    lst = meta[:NBLK, 1]

    xs = _dispatch_scatter(flat, inv0, inv1, CAP)

    yw = pl.pallas_call(
        _ffn_body,
        grid_spec=pltpu.PrefetchScalarGridSpec(
            num_scalar_prefetch=2,
            grid=(NBLK,),
            in_specs=[
                pl.BlockSpec((BLK, D), lambda j, be, ls: (ls[j], 0)),
                pl.BlockSpec((1, D, F), lambda j, be, ls: (be[j], 0, 0)),
                pl.BlockSpec((1, 1, F), lambda j, be, ls: (be[j], 0, 0)),
                pl.BlockSpec((1, F, D), lambda j, be, ls: (be[j], 0, 0)),
                pl.BlockSpec((1, 1, D), lambda j, be, ls: (be[j], 0, 0)),
            ],
            out_specs=pl.BlockSpec((BLK, D), lambda j, be, ls: (ls[j], 0)),
        ),
        out_shape=jax.ShapeDtypeStruct((CAP, D), F32),
    )(bexp, lst, xs, W1, b1.reshape(E, 1, F), W2, b2.reshape(E, 1, D))

    ga, gb = _combine_gather(yw, inv0, inv1)

    out = pl.pallas_call(
        _combine_body,
        out_shape=jax.ShapeDtypeStruct((T, D), F32),
    )(ga, gb, gcol)

    VB = 1280
    logits = pl.pallas_call(
        _vocab_body,
        grid=(V // VB,),
        in_specs=[
            pl.BlockSpec((T, D), lambda v: (0, 0)),
            pl.BlockSpec((D, VB), lambda v: (0, v)),
            pl.BlockSpec((1, VB), lambda v: (0, v)),
        ],
        out_specs=pl.BlockSpec((T, VB), lambda v: (0, v)),
        out_shape=jax.ShapeDtypeStruct((T, V), F32),
    )(out, Wv, bv.reshape(1, V))

    return logits.reshape(Bv, S, V), lb.reshape(())


# speedup vs baseline: 1.4306x; 1.0036x over previous
import functools

import jax
import jax.numpy as jnp
from jax import lax
from jax.experimental import pallas as pl
from jax.experimental.pallas import tpu as pltpu
from jax.experimental.pallas import tpu_sc as plsc

F32 = jnp.float32
BF16 = jnp.bfloat16
BLK = 512
NBLK = 16


def _emb_gather(ids, emb):
    T = ids.shape[0]
    D = emb.shape[1]
    info = plsc.get_sparse_core_info()
    NW = info.num_cores * info.num_subcores
    b_per_w = T // NW
    mesh = plsc.VectorSubcoreMesh(core_axis_name="c", subcore_axis_name="s")

    @functools.partial(
        pl.kernel,
        mesh=mesh,
        out_type=jax.ShapeDtypeStruct((T, D), F32),
        scratch_types=[
            pltpu.VMEM((b_per_w,), jnp.int32),
            pltpu.VMEM((b_per_w, D), F32),
            pltpu.SemaphoreType.DMA,
        ],
    )
    def k(ids_hbm, table_hbm, out_hbm, idx_v, rows_v, sem):
        wid = lax.axis_index("s") * info.num_cores + lax.axis_index("c")
        base = wid * b_per_w
        pltpu.sync_copy(ids_hbm.at[pl.ds(base, b_per_w)], idx_v)
        pltpu.async_copy(table_hbm.at[idx_v], rows_v, sem).wait()
        pltpu.sync_copy(rows_v, out_hbm.at[pl.ds(base, b_per_w)])

    return k(ids, emb)


def _dispatch_scatter(flat, inv0, inv1, cap):
    T, D = flat.shape
    info = plsc.get_sparse_core_info()
    NW = info.num_cores * info.num_subcores
    b_per_w = T // NW
    mesh = plsc.VectorSubcoreMesh(core_axis_name="c", subcore_axis_name="s")

    @functools.partial(
        pl.kernel,
        mesh=mesh,
        out_type=jax.ShapeDtypeStruct((cap, D), F32),
        scratch_types=[
            pltpu.VMEM((b_per_w,), jnp.int32),
            pltpu.VMEM((b_per_w, D), F32),
            pltpu.SemaphoreType.DMA,
        ],
    )
    def k(flat_hbm, inv0_hbm, inv1_hbm, xs_hbm, idx_v, rows_v, sem):
        wid = lax.axis_index("s") * info.num_cores + lax.axis_index("c")
        base = wid * b_per_w
        pltpu.sync_copy(flat_hbm.at[pl.ds(base, b_per_w)], rows_v)
        pltpu.sync_copy(inv0_hbm.at[pl.ds(base, b_per_w)], idx_v)
        pltpu.async_copy(rows_v, xs_hbm.at[idx_v], sem).wait()
        pltpu.sync_copy(inv1_hbm.at[pl.ds(base, b_per_w)], idx_v)
        pltpu.async_copy(rows_v, xs_hbm.at[idx_v], sem).wait()

    return k(flat, inv0, inv1)


def _combine_gather(yw, inv0, inv1, grep):
    T = inv0.shape[0]
    D = yw.shape[1]
    info = plsc.get_sparse_core_info()
    NW = info.num_cores * info.num_subcores
    CH = 32
    b_per_w = T // NW
    n_ch = b_per_w // CH
    mesh = plsc.VectorSubcoreMesh(core_axis_name="c", subcore_axis_name="s")

    @functools.partial(
        pl.kernel,
        mesh=mesh,
        out_type=jax.ShapeDtypeStruct((T, D), F32),
        scratch_types=[
            pltpu.VMEM((CH,), jnp.int32),
            pltpu.VMEM((CH, D), F32),
            pltpu.VMEM((CH, D), F32),
            pltpu.VMEM((CH, 128), F32),
            pltpu.SemaphoreType.DMA,
        ],
    )
    def k(yw_hbm, inv0_hbm, inv1_hbm, grep_hbm, out_hbm,
          idx_v, rows_a, rows_b, g_v, sem):
        wid = lax.axis_index("s") * info.num_cores + lax.axis_index("c")
        for c in range(n_ch):
            base = wid * b_per_w + c * CH
            pltpu.sync_copy(inv0_hbm.at[pl.ds(base, CH)], idx_v)
            pltpu.async_copy(yw_hbm.at[idx_v], rows_a, sem).wait()
            pltpu.sync_copy(inv1_hbm.at[pl.ds(base, CH)], idx_v)
            pltpu.async_copy(yw_hbm.at[idx_v], rows_b, sem).wait()
            pltpu.sync_copy(grep_hbm.at[pl.ds(base, CH)], g_v)

            def row(r, _):
                g0 = g_v[r, 0:16]
                g1 = g_v[r, 16:32]
                for q in range(D // 16):
                    sl = pl.ds(q * 16, 16)
                    rows_a[r, sl] = (g0 * rows_a[r, sl]
                                     + g1 * rows_b[r, sl])
                return 0

            lax.fori_loop(0, CH, row, 0)
            pltpu.sync_copy(rows_a, out_hbm.at[pl.ds(base, CH)])

    return k(yw, inv0, inv1, grep)


def _router_meta_body(flat_ref, wr_ref, invs_ref, gcol_ref, meta_ref, lb_ref):
    flat = flat_ref[...]
    T = flat.shape[0]
    logits = jnp.dot(flat, wr_ref[...], preferred_element_type=F32)
    lane = lax.broadcasted_iota(jnp.int32, logits.shape, 1)
    valid = lane < 8
    logits = jnp.where(valid, logits, -1e30)
    m = jnp.max(logits, axis=1, keepdims=True)
    ex = jnp.where(valid, jnp.exp(logits - m), 0.0)
    probs = ex / jnp.sum(ex, axis=1, keepdims=True)
    pm1 = jnp.where(valid, probs, -1.0)
    m1 = jnp.max(pm1, axis=1, keepdims=True)
    i1 = jnp.min(jnp.where(pm1 == m1, lane, 999), axis=1, keepdims=True)
    is1 = lane == i1
    pm2 = jnp.where(is1, -1.0, pm1)
    m2 = jnp.max(pm2, axis=1, keepdims=True)
    i2 = jnp.min(jnp.where(pm2 == m2, lane, 999), axis=1, keepdims=True)
    is2 = lane == i2
    denom = m1 + m2 + 1e-9
    g1 = m1 / denom
    g2 = m2 / denom
    mask = is1 | is2
    maskf = mask.astype(F32)

    r0 = lax.broadcasted_iota(jnp.int32, (128, 128), 0)
    c0 = lax.broadcasted_iota(jnp.int32, (128, 128), 1)
    lt = (r0 >= c0).astype(F32)
    uts = (r0 < c0).astype(F32)
    chunks = []
    running = jnp.zeros((1, 128), F32)
    for c in range(T // 128):
        blk = maskf[c * 128:(c + 1) * 128, :]
        cumc = jnp.dot(lt, blk, preferred_element_type=F32) + running
        running = cumc[127:128, :]
        chunks.append(cumc)
    cum = jnp.concatenate(chunks, axis=0)
    counts = running
    ci = counts.astype(jnp.int32)
    nb = (ci + (BLK - 1)) // BLK
    nbf = nb.astype(F32)
    baseblk = jnp.dot(nbf, uts, preferred_element_type=F32)
    basef = BLK * baseblk
    pos = basef + cum - 1.0

    colrank = jnp.dot(maskf, uts, preferred_element_type=F32)
    sel0 = maskf * (colrank == 0.0).astype(F32)
    sel1 = maskf * (colrank == 1.0).astype(F32)
    inv0 = jnp.sum(sel0 * pos, axis=1, keepdims=True).astype(jnp.int32)
    inv1 = jnp.sum(sel1 * pos, axis=1, keepdims=True).astype(jnp.int32)
    glow = jnp.where(i1 < i2, g1, g2)
    ghigh = jnp.where(i1 < i2, g2, g1)
    invs_ref[...] = jnp.where(lane == 0, inv0,
                              jnp.where(lane == 1, inv1, 0))
    gcol_ref[...] = jnp.where(lane < 16, glow,
                              jnp.where(lane < 32, ghigh, 0.0))

    jrow = lax.broadcasted_iota(jnp.int32, (32, 128), 0).astype(F32)
    lane32 = lax.broadcasted_iota(jnp.int32, (32, 128), 1)
    bb = jnp.broadcast_to(baseblk, (32, 128))
    nbb = jnp.broadcast_to(nbf, (32, 128))
    active_e = ((jrow >= bb) & (jrow < bb + nbb)).astype(F32)
    bexpf = jnp.sum(active_e * lane32.astype(F32), axis=1, keepdims=True)
    nact = jnp.sum(active_e, axis=1, keepdims=True)
    totb = jnp.sum(nbf)
    le = jnp.max(jnp.where(ci > 0, lane[0:1, :], -1))
    bexp = jnp.where(nact > 0, bexpf, le.astype(F32)).astype(jnp.int32)
    lst = jnp.minimum(jrow[:, 0:1], totb - 1.0).astype(jnp.int32)
    meta_ref[...] = jnp.where(lane32 == 0, bexp,
                              jnp.where(lane32 == 1, lst, 0))

    pmean = jnp.mean(probs, axis=0)
    frac = jnp.mean(maskf, axis=0)
    lb_ref[0, 0] = 8.0 * jnp.sum(frac * pmean)


def _ffn_body(bexp_ref, lst_ref, xs_ref, w1_ref, b1_ref, w2_ref, b2_ref,
              yw_ref):
    j = pl.program_id(0)

    @pl.when(lst_ref[j] == j)
    def _():
        x = xs_ref[...]
        h = jnp.dot(x, w1_ref[0], preferred_element_type=F32) + b1_ref[0]
        h = jax.nn.gelu(h)
        yw_ref[...] = (
            jnp.dot(h, w2_ref[0], preferred_element_type=F32) + b2_ref[0]
        )


def _vocab_body(out_ref, wv_ref, bv_ref, logits_ref):
    logits_ref[...] = (
        jnp.dot(out_ref[...], wv_ref[...], preferred_element_type=F32)
        + bv_ref[...]
    )


def kernel(x, emb, Wr, W1, b1, W2, b2, Wv, bv):
    Bv, S = x.shape
    V, D = emb.shape
    E, _, F = W1.shape
    T = Bv * S
    CAP = NBLK * BLK

    ids = x.reshape(T).astype(jnp.int32)
    flat = _emb_gather(ids, emb)

    Wr_pad = jnp.pad(Wr, ((0, 0), (0, 128 - E)))
    invs, gcol, meta, lb = pl.pallas_call(
        _router_meta_body,
        out_shape=[
            jax.ShapeDtypeStruct((T, 128), jnp.int32),
            jax.ShapeDtypeStruct((T, 128), F32),
            jax.ShapeDtypeStruct((32, 128), jnp.int32),
            jax.ShapeDtypeStruct((1, 1), F32),
        ],
        out_specs=[
            pl.BlockSpec((T, 128), lambda: (0, 0)),
            pl.BlockSpec((T, 128), lambda: (0, 0)),
            pl.BlockSpec((32, 128), lambda: (0, 0)),
            pl.BlockSpec(memory_space=pltpu.SMEM),
        ],
    )(flat, Wr_pad)

    inv0 = invs[:, 0]
    inv1 = invs[:, 1]
    bexp = meta[:NBLK, 0]
    lst = meta[:NBLK, 1]

    xs = _dispatch_scatter(flat, inv0, inv1, CAP)

    yw = pl.pallas_call(
        _ffn_body,
        grid_spec=pltpu.PrefetchScalarGridSpec(
            num_scalar_prefetch=2,
            grid=(NBLK,),
            in_specs=[
                pl.BlockSpec((BLK, D), lambda j, be, ls: (ls[j], 0)),
                pl.BlockSpec((1, D, F), lambda j, be, ls: (be[j], 0, 0)),
                pl.BlockSpec((1, 1, F), lambda j, be, ls: (be[j], 0, 0)),
                pl.BlockSpec((1, F, D), lambda j, be, ls: (be[j], 0, 0)),
                pl.BlockSpec((1, 1, D), lambda j, be, ls: (be[j], 0, 0)),
            ],
            out_specs=pl.BlockSpec((BLK, D), lambda j, be, ls: (ls[j], 0)),
        ),
        out_shape=jax.ShapeDtypeStruct((CAP, D), F32),
    )(bexp, lst, xs, W1, b1.reshape(E, 1, F), W2, b2.reshape(E, 1, D))

    out = _combine_gather(yw, inv0, inv1, gcol)

    VB = 1280
    logits = pl.pallas_call(
        _vocab_body,
        grid=(V // VB,),
        in_specs=[
            pl.BlockSpec((T, D), lambda v: (0, 0)),
            pl.BlockSpec((D, VB), lambda v: (0, v)),
            pl.BlockSpec((1, VB), lambda v: (0, v)),
        ],
        out_specs=pl.BlockSpec((T, VB), lambda v: (0, v)),
        out_shape=jax.ShapeDtypeStruct((T, V), F32),
    )(out, Wv, bv.reshape(1, V))

    return logits.reshape(Bv, S, V), lb.reshape(())


# speedup vs baseline: 1.4524x; 1.0153x over previous
import functools

import jax
import jax.numpy as jnp
from jax import lax
from jax.experimental import pallas as pl
from jax.experimental.pallas import tpu as pltpu
from jax.experimental.pallas import tpu_sc as plsc

F32 = jnp.float32
BF16 = jnp.bfloat16
BLK = 512
NBLK = 16


def _emb_gather(ids, emb):
    T = ids.shape[0]
    D = emb.shape[1]
    info = plsc.get_sparse_core_info()
    NW = info.num_cores * info.num_subcores
    b_per_w = T // NW
    mesh = plsc.VectorSubcoreMesh(core_axis_name="c", subcore_axis_name="s")

    @functools.partial(
        pl.kernel,
        mesh=mesh,
        out_type=jax.ShapeDtypeStruct((T, D), F32),
        scratch_types=[
            pltpu.VMEM((b_per_w,), jnp.int32),
            pltpu.VMEM((b_per_w, D), F32),
            pltpu.SemaphoreType.DMA,
        ],
    )
    def k(ids_hbm, table_hbm, out_hbm, idx_v, rows_v, sem):
        wid = lax.axis_index("s") * info.num_cores + lax.axis_index("c")
        base = wid * b_per_w
        pltpu.sync_copy(ids_hbm.at[pl.ds(base, b_per_w)], idx_v)
        pltpu.async_copy(table_hbm.at[idx_v], rows_v, sem).wait()
        pltpu.sync_copy(rows_v, out_hbm.at[pl.ds(base, b_per_w)])

    return k(ids, emb)


def _dispatch_scatter(flat, inv0, inv1, cap):
    T, D = flat.shape
    info = plsc.get_sparse_core_info()
    NW = info.num_cores * info.num_subcores
    b_per_w = T // NW
    mesh = plsc.VectorSubcoreMesh(core_axis_name="c", subcore_axis_name="s")

    @functools.partial(
        pl.kernel,
        mesh=mesh,
        out_type=jax.ShapeDtypeStruct((cap, D), F32),
        scratch_types=[
            pltpu.VMEM((b_per_w,), jnp.int32),
            pltpu.VMEM((b_per_w, D), F32),
            pltpu.SemaphoreType.DMA,
        ],
    )
    def k(flat_hbm, inv0_hbm, inv1_hbm, xs_hbm, idx_v, rows_v, sem):
        wid = lax.axis_index("s") * info.num_cores + lax.axis_index("c")
        base = wid * b_per_w
        pltpu.sync_copy(flat_hbm.at[pl.ds(base, b_per_w)], rows_v)
        pltpu.sync_copy(inv0_hbm.at[pl.ds(base, b_per_w)], idx_v)
        pltpu.async_copy(rows_v, xs_hbm.at[idx_v], sem).wait()
        pltpu.sync_copy(inv1_hbm.at[pl.ds(base, b_per_w)], idx_v)
        pltpu.async_copy(rows_v, xs_hbm.at[idx_v], sem).wait()

    return k(flat, inv0, inv1)


def _combine_gather(yw, inv0, inv1, grep):
    T = inv0.shape[0]
    D = yw.shape[1]
    info = plsc.get_sparse_core_info()
    NW = info.num_cores * info.num_subcores
    CH = 16
    b_per_w = T // NW
    n_ch = b_per_w // CH
    mesh = plsc.VectorSubcoreMesh(core_axis_name="c", subcore_axis_name="s")

    @functools.partial(
        pl.kernel,
        mesh=mesh,
        out_type=jax.ShapeDtypeStruct((T, D), F32),
        scratch_types=[
            [pltpu.VMEM((CH,), jnp.int32) for _ in range(2)],
            [pltpu.VMEM((CH,), jnp.int32) for _ in range(2)],
            [pltpu.VMEM((CH, D), F32) for _ in range(2)],
            [pltpu.VMEM((CH, D), F32) for _ in range(2)],
            [pltpu.VMEM((CH, 128), F32) for _ in range(2)],
            [pltpu.SemaphoreType.DMA for _ in range(2)],
        ],
    )
    def k(yw_hbm, inv0_hbm, inv1_hbm, grep_hbm, out_hbm,
          idx0, idx1, rows_a, rows_b, g_v, sem):
        wid = lax.axis_index("s") * info.num_cores + lax.axis_index("c")

        def issue(c):
            p = c & 1
            base = wid * b_per_w + c * CH
            pltpu.sync_copy(inv0_hbm.at[pl.ds(base, CH)], idx0[p])
            pltpu.sync_copy(inv1_hbm.at[pl.ds(base, CH)], idx1[p])
            pltpu.sync_copy(grep_hbm.at[pl.ds(base, CH)], g_v[p])
            h1 = pltpu.async_copy(yw_hbm.at[idx0[p]], rows_a[p], sem[p])
            h2 = pltpu.async_copy(yw_hbm.at[idx1[p]], rows_b[p], sem[p])
            return (h1, h2)

        pend = issue(0)
        for c in range(n_ch):
            p = c & 1
            nxt = issue(c + 1) if c + 1 < n_ch else None
            for h in pend:
                h.wait()
            pend = nxt
            ra, rb, gv = rows_a[p], rows_b[p], g_v[p]

            def row(r, _):
                g0 = gv[r, 0:16]
                g1 = gv[r, 16:32]
                for q in range(D // 16):
                    sl = pl.ds(q * 16, 16)
                    ra[r, sl] = g0 * ra[r, sl] + g1 * rb[r, sl]
                return 0

            lax.fori_loop(0, CH, row, 0)
            base = wid * b_per_w + c * CH
            pltpu.sync_copy(ra, out_hbm.at[pl.ds(base, CH)])

    return k(yw, inv0, inv1, grep)


def _router_meta_body(flat_ref, wr_ref, invs_ref, gcol_ref, meta_ref, lb_ref):
    flat = flat_ref[...]
    T = flat.shape[0]
    logits = jnp.dot(flat, wr_ref[...], preferred_element_type=F32)
    lane = lax.broadcasted_iota(jnp.int32, logits.shape, 1)
    valid = lane < 8
    logits = jnp.where(valid, logits, -1e30)
    m = jnp.max(logits, axis=1, keepdims=True)
    ex = jnp.where(valid, jnp.exp(logits - m), 0.0)
    probs = ex / jnp.sum(ex, axis=1, keepdims=True)
    pm1 = jnp.where(valid, probs, -1.0)
    m1 = jnp.max(pm1, axis=1, keepdims=True)
    i1 = jnp.min(jnp.where(pm1 == m1, lane, 999), axis=1, keepdims=True)
    is1 = lane == i1
    pm2 = jnp.where(is1, -1.0, pm1)
    m2 = jnp.max(pm2, axis=1, keepdims=True)
    i2 = jnp.min(jnp.where(pm2 == m2, lane, 999), axis=1, keepdims=True)
    is2 = lane == i2
    denom = m1 + m2 + 1e-9
    g1 = m1 / denom
    g2 = m2 / denom
    mask = is1 | is2
    maskf = mask.astype(F32)

    r0 = lax.broadcasted_iota(jnp.int32, (128, 128), 0)
    c0 = lax.broadcasted_iota(jnp.int32, (128, 128), 1)
    lt = (r0 >= c0).astype(F32)
    uts = (r0 < c0).astype(F32)
    chunks = []
    running = jnp.zeros((1, 128), F32)
    for c in range(T // 128):
        blk = maskf[c * 128:(c + 1) * 128, :]
        cumc = jnp.dot(lt, blk, preferred_element_type=F32) + running
        running = cumc[127:128, :]
        chunks.append(cumc)
    cum = jnp.concatenate(chunks, axis=0)
    counts = running
    ci = counts.astype(jnp.int32)
    nb = (ci + (BLK - 1)) // BLK
    nbf = nb.astype(F32)
    baseblk = jnp.dot(nbf, uts, preferred_element_type=F32)
    basef = BLK * baseblk
    pos = basef + cum - 1.0

    colrank = jnp.dot(maskf, uts, preferred_element_type=F32)
    sel0 = maskf * (colrank == 0.0).astype(F32)
    sel1 = maskf * (colrank == 1.0).astype(F32)
    inv0 = jnp.sum(sel0 * pos, axis=1, keepdims=True).astype(jnp.int32)
    inv1 = jnp.sum(sel1 * pos, axis=1, keepdims=True).astype(jnp.int32)
    glow = jnp.where(i1 < i2, g1, g2)
    ghigh = jnp.where(i1 < i2, g2, g1)
    invs_ref[...] = jnp.where(lane == 0, inv0,
                              jnp.where(lane == 1, inv1, 0))
    gcol_ref[...] = jnp.where(lane < 16, glow,
                              jnp.where(lane < 32, ghigh, 0.0))

    jrow = lax.broadcasted_iota(jnp.int32, (32, 128), 0).astype(F32)
    lane32 = lax.broadcasted_iota(jnp.int32, (32, 128), 1)
    bb = jnp.broadcast_to(baseblk, (32, 128))
    nbb = jnp.broadcast_to(nbf, (32, 128))
    active_e = ((jrow >= bb) & (jrow < bb + nbb)).astype(F32)
    bexpf = jnp.sum(active_e * lane32.astype(F32), axis=1, keepdims=True)
    nact = jnp.sum(active_e, axis=1, keepdims=True)
    totb = jnp.sum(nbf)
    le = jnp.max(jnp.where(ci > 0, lane[0:1, :], -1))
    bexp = jnp.where(nact > 0, bexpf, le.astype(F32)).astype(jnp.int32)
    lst = jnp.minimum(jrow[:, 0:1], totb - 1.0).astype(jnp.int32)
    meta_ref[...] = jnp.where(lane32 == 0, bexp,
                              jnp.where(lane32 == 1, lst, 0))

    pmean = jnp.mean(probs, axis=0)
    frac = jnp.mean(maskf, axis=0)
    lb_ref[0, 0] = 8.0 * jnp.sum(frac * pmean)


def _ffn_body(bexp_ref, lst_ref, xs_ref, w1_ref, b1_ref, w2_ref, b2_ref,
              yw_ref):
    j = pl.program_id(0)

    @pl.when(lst_ref[j] == j)
    def _():
        x = xs_ref[...]
        h = jnp.dot(x, w1_ref[0], preferred_element_type=F32) + b1_ref[0]
        h = jax.nn.gelu(h)
        yw_ref[...] = (
            jnp.dot(h, w2_ref[0], preferred_element_type=F32) + b2_ref[0]
        )


def _vocab_body(out_ref, wv_ref, bv_ref, logits_ref):
    logits_ref[...] = (
        jnp.dot(out_ref[...], wv_ref[...], preferred_element_type=F32)
        + bv_ref[...]
    )


def kernel(x, emb, Wr, W1, b1, W2, b2, Wv, bv):
    Bv, S = x.shape
    V, D = emb.shape
    E, _, F = W1.shape
    T = Bv * S
    CAP = NBLK * BLK

    ids = x.reshape(T).astype(jnp.int32)
    flat = _emb_gather(ids, emb)

    Wr_pad = jnp.pad(Wr, ((0, 0), (0, 128 - E)))
    invs, gcol, meta, lb = pl.pallas_call(
        _router_meta_body,
        out_shape=[
            jax.ShapeDtypeStruct((T, 128), jnp.int32),
            jax.ShapeDtypeStruct((T, 128), F32),
            jax.ShapeDtypeStruct((32, 128), jnp.int32),
            jax.ShapeDtypeStruct((1, 1), F32),
        ],
        out_specs=[
            pl.BlockSpec((T, 128), lambda: (0, 0)),
            pl.BlockSpec((T, 128), lambda: (0, 0)),
            pl.BlockSpec((32, 128), lambda: (0, 0)),
            pl.BlockSpec(memory_space=pltpu.SMEM),
        ],
    )(flat, Wr_pad)

    inv0 = invs[:, 0]
    inv1 = invs[:, 1]
    bexp = meta[:NBLK, 0]
    lst = meta[:NBLK, 1]

    xs = _dispatch_scatter(flat, inv0, inv1, CAP)

    yw = pl.pallas_call(
        _ffn_body,
        grid_spec=pltpu.PrefetchScalarGridSpec(
            num_scalar_prefetch=2,
            grid=(NBLK,),
            in_specs=[
                pl.BlockSpec((BLK, D), lambda j, be, ls: (ls[j], 0)),
                pl.BlockSpec((1, D, F), lambda j, be, ls: (be[j], 0, 0)),
                pl.BlockSpec((1, 1, F), lambda j, be, ls: (be[j], 0, 0)),
                pl.BlockSpec((1, F, D), lambda j, be, ls: (be[j], 0, 0)),
                pl.BlockSpec((1, 1, D), lambda j, be, ls: (be[j], 0, 0)),
            ],
            out_specs=pl.BlockSpec((BLK, D), lambda j, be, ls: (ls[j], 0)),
        ),
        out_shape=jax.ShapeDtypeStruct((CAP, D), F32),
    )(bexp, lst, xs, W1, b1.reshape(E, 1, F), W2, b2.reshape(E, 1, D))

    out = _combine_gather(yw, inv0, inv1, gcol)

    VB = 1280
    logits = pl.pallas_call(
        _vocab_body,
        grid=(V // VB,),
        in_specs=[
            pl.BlockSpec((T, D), lambda v: (0, 0)),
            pl.BlockSpec((D, VB), lambda v: (0, v)),
            pl.BlockSpec((1, VB), lambda v: (0, v)),
        ],
        out_specs=pl.BlockSpec((T, VB), lambda v: (0, v)),
        out_shape=jax.ShapeDtypeStruct((T, V), F32),
    )(out, Wv, bv.reshape(1, V))

    return logits.reshape(Bv, S, V), lb.reshape(())


# speedup vs baseline: 1.4817x; 1.0201x over previous
import functools

import jax
import jax.numpy as jnp
from jax import lax
from jax.experimental import pallas as pl
from jax.experimental.pallas import tpu as pltpu
from jax.experimental.pallas import tpu_sc as plsc

F32 = jnp.float32
BF16 = jnp.bfloat16
BLK = 512
NBLK = 16


def _emb_gather(ids, emb):
    T = ids.shape[0]
    D = emb.shape[1]
    info = plsc.get_sparse_core_info()
    NW = info.num_cores * info.num_subcores
    b_per_w = T // NW
    mesh = plsc.VectorSubcoreMesh(core_axis_name="c", subcore_axis_name="s")

    @functools.partial(
        pl.kernel,
        mesh=mesh,
        out_type=jax.ShapeDtypeStruct((T, D), F32),
        scratch_types=[
            pltpu.VMEM((b_per_w,), jnp.int32),
            pltpu.VMEM((b_per_w, D), F32),
            pltpu.SemaphoreType.DMA,
        ],
    )
    def k(ids_hbm, table_hbm, out_hbm, idx_v, rows_v, sem):
        wid = lax.axis_index("s") * info.num_cores + lax.axis_index("c")
        base = wid * b_per_w
        pltpu.sync_copy(ids_hbm.at[pl.ds(base, b_per_w)], idx_v)
        pltpu.async_copy(table_hbm.at[idx_v], rows_v, sem).wait()
        pltpu.sync_copy(rows_v, out_hbm.at[pl.ds(base, b_per_w)])

    return k(ids, emb)


def _dispatch_scatter(flat, inv0, inv1, cap):
    T, D = flat.shape
    info = plsc.get_sparse_core_info()
    NW = info.num_cores * info.num_subcores
    b_per_w = T // NW
    mesh = plsc.VectorSubcoreMesh(core_axis_name="c", subcore_axis_name="s")

    @functools.partial(
        pl.kernel,
        mesh=mesh,
        out_type=jax.ShapeDtypeStruct((cap, D), jnp.int32),
        scratch_types=[
            pltpu.VMEM((b_per_w,), jnp.int32),
            pltpu.VMEM((b_per_w, D), jnp.int32),
            pltpu.SemaphoreType.DMA,
        ],
    )
    def k(flat_hbm, inv0_hbm, inv1_hbm, xs_hbm, idx_v, rows_v, sem):
        wid = lax.axis_index("s") * info.num_cores + lax.axis_index("c")
        base = wid * b_per_w
        pltpu.sync_copy(flat_hbm.at[pl.ds(base, b_per_w)], rows_v)
        pltpu.sync_copy(inv0_hbm.at[pl.ds(base, b_per_w)], idx_v)
        pltpu.async_copy(rows_v, xs_hbm.at[idx_v], sem).wait()
        pltpu.sync_copy(inv1_hbm.at[pl.ds(base, b_per_w)], idx_v)
        pltpu.async_copy(rows_v, xs_hbm.at[idx_v], sem).wait()

    return k(flat, inv0, inv1)


def _combine_gather(yw, inv0, inv1, grep):
    T = inv0.shape[0]
    D = yw.shape[1]
    info = plsc.get_sparse_core_info()
    NW = info.num_cores * info.num_subcores
    CH = 16
    b_per_w = T // NW
    n_ch = b_per_w // CH
    mesh = plsc.VectorSubcoreMesh(core_axis_name="c", subcore_axis_name="s")

    @functools.partial(
        pl.kernel,
        mesh=mesh,
        out_type=jax.ShapeDtypeStruct((T, D), F32),
        scratch_types=[
            [pltpu.VMEM((CH,), jnp.int32) for _ in range(2)],
            [pltpu.VMEM((CH,), jnp.int32) for _ in range(2)],
            [pltpu.VMEM((CH, D), F32) for _ in range(2)],
            [pltpu.VMEM((CH, D), F32) for _ in range(2)],
            [pltpu.VMEM((CH, 128), F32) for _ in range(2)],
            [pltpu.SemaphoreType.DMA for _ in range(2)],
        ],
    )
    def k(yw_hbm, inv0_hbm, inv1_hbm, grep_hbm, out_hbm,
          idx0, idx1, rows_a, rows_b, g_v, sem):
        wid = lax.axis_index("s") * info.num_cores + lax.axis_index("c")

        def issue(c):
            p = c & 1
            base = wid * b_per_w + c * CH
            pltpu.sync_copy(inv0_hbm.at[pl.ds(base, CH)], idx0[p])
            pltpu.sync_copy(inv1_hbm.at[pl.ds(base, CH)], idx1[p])
            pltpu.sync_copy(grep_hbm.at[pl.ds(base, CH)], g_v[p])
            h1 = pltpu.async_copy(yw_hbm.at[idx0[p]], rows_a[p], sem[p])
            h2 = pltpu.async_copy(yw_hbm.at[idx1[p]], rows_b[p], sem[p])
            return (h1, h2)

        pend = issue(0)
        for c in range(n_ch):
            p = c & 1
            nxt = issue(c + 1) if c + 1 < n_ch else None
            for h in pend:
                h.wait()
            pend = nxt
            ra, rb, gv = rows_a[p], rows_b[p], g_v[p]

            def row(r, _):
                g0 = gv[r, 0:16]
                g1 = gv[r, 16:32]
                for q in range(D // 16):
                    sl = pl.ds(q * 16, 16)
                    ra[r, sl] = g0 * ra[r, sl] + g1 * rb[r, sl]
                return 0

            lax.fori_loop(0, CH, row, 0)
            base = wid * b_per_w + c * CH
            pltpu.sync_copy(ra, out_hbm.at[pl.ds(base, CH)])

    return k(yw, inv0, inv1, grep)


def _router_meta_body(flat_ref, wr_ref, invs_ref, gcol_ref, meta_ref, pk_ref,
                      lb_ref):
    flat = flat_ref[...]
    T = flat.shape[0]
    logits = jnp.dot(flat, wr_ref[...], preferred_element_type=F32)
    lane = lax.broadcasted_iota(jnp.int32, logits.shape, 1)
    valid = lane < 8
    logits = jnp.where(valid, logits, -1e30)
    m = jnp.max(logits, axis=1, keepdims=True)
    ex = jnp.where(valid, jnp.exp(logits - m), 0.0)
    probs = ex / jnp.sum(ex, axis=1, keepdims=True)
    pm1 = jnp.where(valid, probs, -1.0)
    m1 = jnp.max(pm1, axis=1, keepdims=True)
    i1 = jnp.min(jnp.where(pm1 == m1, lane, 999), axis=1, keepdims=True)
    is1 = lane == i1
    pm2 = jnp.where(is1, -1.0, pm1)
    m2 = jnp.max(pm2, axis=1, keepdims=True)
    i2 = jnp.min(jnp.where(pm2 == m2, lane, 999), axis=1, keepdims=True)
    is2 = lane == i2
    denom = m1 + m2 + 1e-9
    g1 = m1 / denom
    g2 = m2 / denom
    mask = is1 | is2
    maskf = mask.astype(F32)

    r0 = lax.broadcasted_iota(jnp.int32, (128, 128), 0)
    c0 = lax.broadcasted_iota(jnp.int32, (128, 128), 1)
    lt = (r0 >= c0).astype(F32)
    uts = (r0 < c0).astype(F32)
    chunks = []
    running = jnp.zeros((1, 128), F32)
    for c in range(T // 128):
        blk = maskf[c * 128:(c + 1) * 128, :]
        cumc = jnp.dot(lt, blk, preferred_element_type=F32) + running
        running = cumc[127:128, :]
        chunks.append(cumc)
    cum = jnp.concatenate(chunks, axis=0)
    counts = running
    ci = counts.astype(jnp.int32)
    nb = (ci + (BLK - 1)) // BLK
    nbf = nb.astype(F32)
    baseblk = jnp.dot(nbf, uts, preferred_element_type=F32)
    basef = BLK * baseblk
    pos = basef + cum - 1.0

    colrank = jnp.dot(maskf, uts, preferred_element_type=F32)
    sel0 = maskf * (colrank == 0.0).astype(F32)
    sel1 = maskf * (colrank == 1.0).astype(F32)
    inv0 = jnp.sum(sel0 * pos, axis=1, keepdims=True).astype(jnp.int32)
    inv1 = jnp.sum(sel1 * pos, axis=1, keepdims=True).astype(jnp.int32)
    glow = jnp.where(i1 < i2, g1, g2)
    ghigh = jnp.where(i1 < i2, g2, g1)
    invs_ref[...] = jnp.where(lane == 0, inv0,
                              jnp.where(lane == 1, inv1, 0))
    gcol_ref[...] = jnp.where(lane < 16, glow,
                              jnp.where(lane < 32, ghigh, 0.0))

    jrow = lax.broadcasted_iota(jnp.int32, (32, 128), 0).astype(F32)
    lane32 = lax.broadcasted_iota(jnp.int32, (32, 128), 1)
    bb = jnp.broadcast_to(baseblk, (32, 128))
    nbb = jnp.broadcast_to(nbf, (32, 128))
    active_e = ((jrow >= bb) & (jrow < bb + nbb)).astype(F32)
    bexpf = jnp.sum(active_e * lane32.astype(F32), axis=1, keepdims=True)
    nact = jnp.sum(active_e, axis=1, keepdims=True)
    totb = jnp.sum(nbf)
    le = jnp.max(jnp.where(ci > 0, lane[0:1, :], -1))
    bexp = jnp.where(nact > 0, bexpf, le.astype(F32)).astype(jnp.int32)
    lst = jnp.minimum(jrow[:, 0:1], totb - 1.0).astype(jnp.int32)
    meta_ref[...] = jnp.where(lane32 == 0, bexp,
                              jnp.where(lane32 == 1, lst, 0))

    Dh = flat.shape[1] // 2
    au = lax.bitcast_convert_type(flat[:, :Dh].astype(jnp.bfloat16),
                                  jnp.uint16)
    bu = lax.bitcast_convert_type(flat[:, Dh:].astype(jnp.bfloat16),
                                  jnp.uint16)
    pk = au.astype(jnp.uint32) | (bu.astype(jnp.uint32) << 16)
    pk_ref[...] = lax.bitcast_convert_type(pk, jnp.int32)

    pmean = jnp.mean(probs, axis=0)
    frac = jnp.mean(maskf, axis=0)
    lb_ref[0, 0] = 8.0 * jnp.sum(frac * pmean)


def _ffn_body(bexp_ref, lst_ref, xs_ref, w1_ref, b1_ref, w2_ref, b2_ref,
              yw_ref):
    j = pl.program_id(0)

    @pl.when(lst_ref[j] == j)
    def _():
        pk = lax.bitcast_convert_type(xs_ref[...], jnp.uint32)
        lo = lax.bitcast_convert_type((pk & 0xFFFF).astype(jnp.uint16),
                                      jnp.bfloat16)
        hi = lax.bitcast_convert_type((pk >> 16).astype(jnp.uint16),
                                      jnp.bfloat16)
        x = jnp.concatenate([lo, hi], axis=1).astype(F32)
        h = jnp.dot(x, w1_ref[0], preferred_element_type=F32) + b1_ref[0]
        h = jax.nn.gelu(h)
        yw_ref[...] = (
            jnp.dot(h, w2_ref[0], preferred_element_type=F32) + b2_ref[0]
        )


def _vocab_body(out_ref, wv_ref, bv_ref, logits_ref):
    logits_ref[...] = (
        jnp.dot(out_ref[...], wv_ref[...], preferred_element_type=F32)
        + bv_ref[...]
    )


def kernel(x, emb, Wr, W1, b1, W2, b2, Wv, bv):
    Bv, S = x.shape
    V, D = emb.shape
    E, _, F = W1.shape
    T = Bv * S
    CAP = NBLK * BLK

    ids = x.reshape(T).astype(jnp.int32)
    flat = _emb_gather(ids, emb)

    Wr_pad = jnp.pad(Wr, ((0, 0), (0, 128 - E)))
    invs, gcol, meta, flat_pk, lb = pl.pallas_call(
        _router_meta_body,
        out_shape=[
            jax.ShapeDtypeStruct((T, 128), jnp.int32),
            jax.ShapeDtypeStruct((T, 128), F32),
            jax.ShapeDtypeStruct((32, 128), jnp.int32),
            jax.ShapeDtypeStruct((T, D // 2), jnp.int32),
            jax.ShapeDtypeStruct((1, 1), F32),
        ],
        out_specs=[
            pl.BlockSpec((T, 128), lambda: (0, 0)),
            pl.BlockSpec((T, 128), lambda: (0, 0)),
            pl.BlockSpec((32, 128), lambda: (0, 0)),
            pl.BlockSpec((T, D // 2), lambda: (0, 0)),
            pl.BlockSpec(memory_space=pltpu.SMEM),
        ],
    )(flat, Wr_pad)

    inv0 = invs[:, 0]
    inv1 = invs[:, 1]
    bexp = meta[:NBLK, 0]
    lst = meta[:NBLK, 1]

    xs = _dispatch_scatter(flat_pk, inv0, inv1, CAP)

    yw = pl.pallas_call(
        _ffn_body,
        grid_spec=pltpu.PrefetchScalarGridSpec(
            num_scalar_prefetch=2,
            grid=(NBLK,),
            in_specs=[
                pl.BlockSpec((BLK, D // 2), lambda j, be, ls: (ls[j], 0)),
                pl.BlockSpec((1, D, F), lambda j, be, ls: (be[j], 0, 0)),
                pl.BlockSpec((1, 1, F), lambda j, be, ls: (be[j], 0, 0)),
                pl.BlockSpec((1, F, D), lambda j, be, ls: (be[j], 0, 0)),
                pl.BlockSpec((1, 1, D), lambda j, be, ls: (be[j], 0, 0)),
            ],
            out_specs=pl.BlockSpec((BLK, D), lambda j, be, ls: (ls[j], 0)),
        ),
        out_shape=jax.ShapeDtypeStruct((CAP, D), F32),
    )(bexp, lst, xs, W1, b1.reshape(E, 1, F), W2, b2.reshape(E, 1, D))

    out = _combine_gather(yw, inv0, inv1, gcol)

    VB = 1280
    logits = pl.pallas_call(
        _vocab_body,
        grid=(V // VB,),
        in_specs=[
            pl.BlockSpec((T, D), lambda v: (0, 0)),
            pl.BlockSpec((D, VB), lambda v: (0, v)),
            pl.BlockSpec((1, VB), lambda v: (0, v)),
        ],
        out_specs=pl.BlockSpec((T, VB), lambda v: (0, v)),
        out_shape=jax.ShapeDtypeStruct((T, V), F32),
    )(out, Wv, bv.reshape(1, V))

    return logits.reshape(Bv, S, V), lb.reshape(())
